# baseline (speedup 1.0000x reference)
import functools
import math

import jax
import jax.numpy as jnp
from jax.experimental import pallas as pl
from jax.experimental.pallas import tpu as pltpu


_BN_SCALE = 1.0 / math.sqrt(1.0 + 1e-5)


def _gc_dense_body(a_ref, w_ref, b_ref, o_ref):
    a = a_ref[...]
    o = jnp.dot(a, w_ref[...].T, preferred_element_type=jnp.float32)
    o = o + b_ref[...]
    o = jnp.maximum(o, 0.0)
    bm, f = o.shape
    o_ref[...] = o.reshape(bm // 4, 4, f).max(axis=1)


def _gc_dense(A, W, b, bm):
    M, ck = A.shape
    F = W.shape[0]
    return pl.pallas_call(
        _gc_dense_body,
        grid=(M // bm,),
        in_specs=[
            pl.BlockSpec((bm, ck), lambda i: (i, 0)),
            pl.BlockSpec((F, ck), lambda i: (0, 0)),
            pl.BlockSpec((1, F), lambda i: (0, 0)),
        ],
        out_specs=pl.BlockSpec((bm // 4, F), lambda i: (i, 0)),
        out_shape=jax.ShapeDtypeStruct((M // 4, F), jnp.float32),
    )(A, W, b.reshape(1, F))


def _fc_body(h_ref, w1_ref, b1_ref, w2t_ref, b2_ref, o_ref, acc_ref):
    i = pl.program_id(0)

    @pl.when(i == 0)
    def _init():
        acc_ref[...] = jnp.zeros_like(acc_ref)

    p = jnp.dot(h_ref[...], w1_ref[...].T, preferred_element_type=jnp.float32)
    s = jax.nn.sigmoid(p + b1_ref[0])
    acc_ref[...] += jnp.dot(s, w2t_ref[...], preferred_element_type=jnp.float32)

    @pl.when(i == pl.num_programs(0) - 1)
    def _fin():
        o_ref[...] = acc_ref[...] + b2_ref[...]


def _fc_fused(H, W1, b1, W2, b2, fblk=64):
    Bb, K = H.shape
    F1 = W1.shape[0]
    F2 = W2.shape[0]
    return pl.pallas_call(
        _fc_body,
        grid=(F1 // fblk,),
        in_specs=[
            pl.BlockSpec((Bb, K), lambda i: (0, 0)),
            pl.BlockSpec((fblk, K), lambda i: (i, 0)),
            pl.BlockSpec((1, 1, fblk), lambda i: (i, 0, 0)),
            pl.BlockSpec((fblk, F2), lambda i: (i, 0)),
            pl.BlockSpec((Bb, F2), lambda i: (0, 0)),
        ],
        out_specs=pl.BlockSpec((Bb, F2), lambda i: (0, 0)),
        out_shape=jax.ShapeDtypeStruct((Bb, F2), jnp.float32),
        scratch_shapes=[pltpu.VMEM((Bb, F2), jnp.float32)],
    )(H, W1, b1.reshape(F1 // fblk, 1, fblk), W2.T,
      jnp.broadcast_to(b2.reshape(1, F2), (Bb, F2)))


def _spmm(rows, cols, vals, X, n):
    return jax.ops.segment_sum(vals[:, None] * jnp.take(X, cols, axis=0),
                               rows, num_segments=n)


def _cheb_inputs(x0, rows, cols, vals, n):
    x1 = _spmm(rows, cols, vals, x0, n)
    x2 = 2.0 * _spmm(rows, cols, vals, x1, n) - x0
    return x0, x1, x2


def kernel(x, rows1, cols1, vals1, rows2, cols2, vals2,
           GCL1_w, GCL1_b, GCL2_w, GCL2_b, FC1_w, FC1_b, FC2_w, FC2_b):
    Bb, Cc, Vv = x.shape
    W1 = GCL1_w * _BN_SCALE

    x0 = jnp.transpose(x, (2, 1, 0)).reshape(Vv, Cc * Bb)
    xs = _cheb_inputs(x0, rows1, cols1, vals1, Vv)
    X = jnp.stack(xs, axis=0).reshape(3, Vv, Cc, Bb)
    X = jnp.transpose(X, (3, 1, 2, 0)).reshape(Bb * Vv, Cc * 3)
    F1 = W1.shape[0]
    h = _gc_dense(X, W1, GCL1_b, bm=6400)
    V2 = Vv // 4
    h = h.reshape(Bb, V2, F1)

    x0b = jnp.transpose(h, (1, 2, 0)).reshape(V2, F1 * Bb)
    xsb = _cheb_inputs(x0b, rows2, cols2, vals2, V2)
    Xb = jnp.stack(xsb, axis=0).reshape(3, V2, F1, Bb)
    Xb = jnp.transpose(Xb, (3, 1, 2, 0)).reshape(Bb * V2, F1 * 3)
    F2v = GCL2_w.shape[0]
    h2 = _gc_dense(Xb, GCL2_w, GCL2_b, bm=6400)
    V4 = V2 // 4
    h2 = h2.reshape(Bb, V4, F2v)

    hf = jnp.transpose(h2, (0, 2, 1)).reshape(Bb, F2v * V4)
    return _fc_fused(hf, FC1_w, FC1_b, FC2_w, FC2_b)


# speedup vs baseline: 1.0000x; 1.0000x over previous
import functools
import math

import jax
import jax.numpy as jnp
from jax import lax
from jax.experimental import pallas as pl
from jax.experimental.pallas import tpu as pltpu
from jax.experimental.pallas import tpu_sc as plsc

_NSUB = 16
_NCORE = 2


_BN_SCALE = 1.0 / math.sqrt(1.0 + 1e-5)


def _gc_dense_body(a_ref, w_ref, b_ref, o_ref):
    a = a_ref[...]
    o = jnp.dot(a, w_ref[...].T, preferred_element_type=jnp.float32)
    o = o + b_ref[...]
    o = jnp.maximum(o, 0.0)
    bm, f = o.shape
    o_ref[...] = o.reshape(bm // 4, 4, f).max(axis=1)


def _gc_dense(A, W, b, bm):
    M, ck = A.shape
    F = W.shape[0]
    return pl.pallas_call(
        _gc_dense_body,
        grid=(M // bm,),
        in_specs=[
            pl.BlockSpec((bm, ck), lambda i: (i, 0)),
            pl.BlockSpec((F, ck), lambda i: (0, 0)),
            pl.BlockSpec((1, F), lambda i: (0, 0)),
        ],
        out_specs=pl.BlockSpec((bm // 4, F), lambda i: (i, 0)),
        out_shape=jax.ShapeDtypeStruct((M // 4, F), jnp.float32),
    )(A, W, b.reshape(1, F))


def _fc_body(h_ref, w1_ref, b1_ref, w2t_ref, b2_ref, o_ref, acc_ref):
    i = pl.program_id(0)

    @pl.when(i == 0)
    def _init():
        acc_ref[...] = jnp.zeros_like(acc_ref)

    p = jnp.dot(h_ref[...], w1_ref[...].T, preferred_element_type=jnp.float32)
    s = jax.nn.sigmoid(p + b1_ref[0])
    acc_ref[...] += jnp.dot(s, w2t_ref[...], preferred_element_type=jnp.float32)

    @pl.when(i == pl.num_programs(0) - 1)
    def _fin():
        o_ref[...] = acc_ref[...] + b2_ref[...]


def _fc_fused(H, W1, b1, W2, b2, fblk=64):
    Bb, K = H.shape
    F1 = W1.shape[0]
    F2 = W2.shape[0]
    return pl.pallas_call(
        _fc_body,
        grid=(F1 // fblk,),
        in_specs=[
            pl.BlockSpec((Bb, K), lambda i: (0, 0)),
            pl.BlockSpec((fblk, K), lambda i: (i, 0)),
            pl.BlockSpec((1, 1, fblk), lambda i: (i, 0, 0)),
            pl.BlockSpec((fblk, F2), lambda i: (i, 0)),
            pl.BlockSpec((Bb, F2), lambda i: (0, 0)),
        ],
        out_specs=pl.BlockSpec((Bb, F2), lambda i: (0, 0)),
        out_shape=jax.ShapeDtypeStruct((Bb, F2), jnp.float32),
        scratch_shapes=[pltpu.VMEM((Bb, F2), jnp.float32)],
    )(H, W1, b1.reshape(F1 // fblk, 1, fblk), W2.T,
      jnp.broadcast_to(b2.reshape(1, F2), (Bb, F2)))


def _bcast_lane(v16, l):
    idx = jnp.full((16, 1), l, jnp.int32)
    return lax.gather(
        v16, idx,
        lax.GatherDimensionNumbers(offset_dims=(), collapsed_slice_dims=(0,),
                                   start_index_map=(0,)),
        slice_sizes=(1,), mode=lax.GatherScatterMode.PROMISE_IN_BOUNDS)


def _spmm_sc_kernel(n_parts, Vp, Dp, n_chunks, ch, n_stat):
    ppc = n_parts // _NCORE
    vt = Vp // _NSUB
    mesh = plsc.VectorSubcoreMesh(core_axis_name="c", subcore_axis_name="s")

    @functools.partial(
        pl.kernel, mesh=mesh,
        out_type=jax.ShapeDtypeStruct((n_parts, Vp, Dp), jnp.float32),
        scratch_types=[
            pltpu.VMEM((ch,), jnp.int32),
            pltpu.VMEM((ch,), jnp.int32),
            pltpu.VMEM((ch // 16, 16), jnp.float32),
            pltpu.VMEM((ch, Dp), jnp.float32),
            pltpu.VMEM_SHARED((Vp, Dp), jnp.float32),
            pltpu.SemaphoreType.DMA,
        ],
    )
    def k(xp, cols_t, rows_t, vals_t, out,
          cols_c, rows_c, vals_c, gbuf, acc, sem):
        cid = lax.axis_index("c")
        sid = lax.axis_index("s")
        zero16 = jnp.zeros((16,), jnp.float32)
        for q in range(ppc):
            p = cid * ppc + q

            def zrow(i, c):
                for d in range(Dp // 16):
                    gbuf[i, pl.ds(d * 16, 16)] = zero16
                return c

            lax.fori_loop(0, ch, zrow, 0)
            off = 0
            while off < vt:
                r = min(ch, vt - off)
                pltpu.sync_copy(gbuf.at[pl.ds(0, r)],
                                acc.at[pl.ds(sid * vt + off, r)])
                off += r
            plsc.subcore_barrier()

            def chunk_body(j, carry):
                pltpu.sync_copy(cols_t.at[sid, j], cols_c)
                pltpu.sync_copy(rows_t.at[sid, j], rows_c)
                pltpu.sync_copy(vals_t.at[sid, j], vals_c)
                poff = jnp.full((16,), p * n_stat, jnp.int32)
                for g in range(ch // 16):
                    sl = pl.ds(g * 16, 16)
                    cols_c[sl] = cols_c[sl] + poff
                pltpu.async_copy(xp.at[cols_c], gbuf, sem).wait()

                def group_body(g, c2):
                    vals16 = vals_c[g]
                    for l in range(16):
                        vv = _bcast_lane(vals16, l)
                        i = g * 16 + l
                        for d in range(Dp // 16):
                            sl = pl.ds(d * 16, 16)
                            gbuf[i, sl] = gbuf[i, sl] * vv
                    return c2

                lax.fori_loop(0, ch // 16, group_body, 0)
                pltpu.sync_copy(gbuf, acc.at[rows_c], add=True)
                return carry

            lax.fori_loop(0, n_chunks, chunk_body, 0)
            plsc.subcore_barrier()
            pltpu.sync_copy(acc.at[pl.ds(sid * vt, vt)],
                            out.at[p, pl.ds(sid * vt, vt)])
            plsc.subcore_barrier()

    return k


def _make_sc_spmm(rows, cols, vals, n, D, n_parts, ch=128):
    E = rows.shape[0]
    Dp = D // n_parts
    Vp = -(-n // (_NSUB * 8)) * (_NSUB * 8)
    et = -(-E // _NSUB)
    n_chunks = -(-et // ch)
    ep = _NSUB * n_chunks * ch
    pad = ep - E
    rows_p = jnp.pad(rows, (0, pad)).reshape(_NSUB, n_chunks, ch)
    vals_p = jnp.pad(vals, (0, pad)).reshape(_NSUB, n_chunks, ch // 16, 16)
    cols_p = jnp.pad(cols, (0, pad)).reshape(_NSUB, n_chunks, ch)
    kfn = _spmm_sc_kernel(n_parts, Vp, Dp, n_chunks, ch, n)

    def spmm(X):
        xp = jnp.transpose(X.reshape(n, n_parts, Dp), (1, 0, 2))
        xp = xp.reshape(n_parts * n, Dp)
        out = kfn(xp, cols_p, rows_p, vals_p)
        out = jnp.transpose(out[:, :n, :], (1, 0, 2)).reshape(n, D)
        return out

    return spmm


def _spmm_xla(rows, cols, vals, X, n):
    return jax.ops.segment_sum(vals[:, None] * jnp.take(X, cols, axis=0),
                               rows, num_segments=n)


def _cheb_inputs(x0, rows, cols, vals, n, n_parts):
    del n_parts
    x1 = _spmm_xla(rows, cols, vals, x0, n)
    x2 = 2.0 * _spmm_xla(rows, cols, vals, x1, n) - x0
    return x0, x1, x2


def kernel(x, rows1, cols1, vals1, rows2, cols2, vals2,
           GCL1_w, GCL1_b, GCL2_w, GCL2_b, FC1_w, FC1_b, FC2_w, FC2_b):
    Bb, Cc, Vv = x.shape
    W1 = GCL1_w * _BN_SCALE

    x0 = jnp.transpose(x, (2, 1, 0)).reshape(Vv, Cc * Bb)
    xs = _cheb_inputs(x0, rows1, cols1, vals1, Vv, n_parts=4)
    X = jnp.stack(xs, axis=0).reshape(3, Vv, Cc, Bb)
    X = jnp.transpose(X, (3, 1, 2, 0)).reshape(Bb * Vv, Cc * 3)
    F1 = W1.shape[0]
    h = _gc_dense(X, W1, GCL1_b, bm=6400)
    V2 = Vv // 4
    h = h.reshape(Bb, V2, F1)

    x0b = jnp.transpose(h, (1, 2, 0)).reshape(V2, F1 * Bb)
    xsb = _cheb_inputs(x0b, rows2, cols2, vals2, V2, n_parts=8)
    Xb = jnp.stack(xsb, axis=0).reshape(3, V2, F1, Bb)
    Xb = jnp.transpose(Xb, (3, 1, 2, 0)).reshape(Bb * V2, F1 * 3)
    F2v = GCL2_w.shape[0]
    h2 = _gc_dense(Xb, GCL2_w, GCL2_b, bm=6400)
    V4 = V2 // 4
    h2 = h2.reshape(Bb, V4, F2v)

    hf = jnp.transpose(h2, (0, 2, 1)).reshape(Bb, F2v * V4)
    return _fc_fused(hf, FC1_w, FC1_b, FC2_w, FC2_b)
